# Initial kernel scaffold; baseline (speedup 1.0000x reference)
import jax
import jax.numpy as jnp
from jax.experimental import pallas as pl


def kernel(x, edge_index, edge_attr, batch, We1, be1, We2, be2, We3, be3, Wc1, bc1, g1, t1, Wc2, bc2, g2, t2, Wc3, bc3, g3, t3, Wr, br, Wk1, bk1, Wk2, bk2):
    raise NotImplementedError("write your pallas kernel here")


# speedup vs baseline: 1.0004x; 1.0004x over previous
import jax
import jax.numpy as jnp
from jax.experimental import pallas as pl

BN_EPS = 1e-5


def _cls_body(z_ref, w1_ref, b1_ref, w2_ref, b2_ref, out_ref):
    z = z_ref[...]
    h = jnp.maximum(jnp.dot(z, w1_ref[...], preferred_element_type=jnp.float32)
                    + b1_ref[...], 0.0)
    out_ref[...] = jnp.dot(h, w2_ref[...], preferred_element_type=jnp.float32) + b2_ref[...]


def _classifier(z, Wk1, bk1, Wk2, bk2):
    G = z.shape[0]
    C = Wk2.shape[1]
    return pl.pallas_call(
        _cls_body,
        out_shape=jax.ShapeDtypeStruct((G, C), jnp.float32),
    )(z, Wk1, bk1.reshape(1, -1), Wk2, bk2.reshape(1, -1))


def _bn(h, gamma, beta):
    mu = jnp.mean(h, axis=0)
    var = jnp.var(h, axis=0)
    return (h - mu) / jnp.sqrt(var + BN_EPS) * gamma + beta


def _gine(x, src, dst, e, W, b, gamma, beta):
    msg = jax.nn.relu(x[src] + e)
    aggr = jax.ops.segment_sum(msg, dst, num_segments=x.shape[0])
    h = (x + aggr) @ W + b
    return jax.nn.relu(_bn(h, gamma, beta))


def kernel(x, edge_index, edge_attr, batch,
           We1, be1, We2, be2, We3, be3,
           Wc1, bc1, g1, t1, Wc2, bc2, g2, t2, Wc3, bc3, g3, t3,
           Wr, br, Wk1, bk1, Wk2, bk2):
    G = 256
    src = edge_index[0]
    dst = edge_index[1]
    e1 = edge_attr @ We1 + be1
    e2 = edge_attr @ We2 + be2
    e3 = edge_attr @ We3 + be3
    h = _gine(x, src, dst, e1, Wc1, bc1, g1, t1) + (x @ Wr + br)
    h = _gine(h, src, dst, e2, Wc2, bc2, g2, t2) + h
    h = _gine(h, src, dst, e3, Wc3, bc3, g3, t3) + h
    ones = jnp.ones((h.shape[0],), dtype=h.dtype)
    counts = jnp.maximum(jax.ops.segment_sum(ones, batch, num_segments=G), 1.0)
    mean_pool = jax.ops.segment_sum(h, batch, num_segments=G) / counts[:, None]
    max_pool = jax.ops.segment_max(h, batch, num_segments=G)
    max_pool = jnp.where(jnp.isfinite(max_pool), max_pool, 0.0)
    z = jnp.concatenate([mean_pool, max_pool], axis=1)
    return _classifier(z, Wk1, bk1, Wk2, bk2)


# speedup vs baseline: 3.2320x; 3.2305x over previous
import functools

import jax
import jax.numpy as jnp
from jax import lax
from jax.experimental import pallas as pl
from jax.experimental.pallas import tpu as pltpu
from jax.experimental.pallas import tpu_sc as plsc

N = 50000
NP = 51200
E = 800000
ER = E // 128
G = 256
GP = 272
BN_EPS = 1e-5
BLK = 1024
NBLK = NP // BLK

_mesh = plsc.VectorSubcoreMesh(core_axis_name="c", subcore_axis_name="s")


def _sc_msg16_body(table, eidx, eattr, wpk, out,
                   src_v, dst_v, ea_v, rows_v, w_v, zb_v, aggr, sem):
    c = lax.axis_index("c")
    s = lax.axis_index("s")
    wid = s * 2 + c

    pltpu.sync_copy(wpk, w_v)
    w0 = w_v[0, pl.ds(0, 16)]
    w1 = w_v[1, pl.ds(0, 16)]
    bb = w_v[2, pl.ds(0, 16)]
    z16 = jnp.zeros((16,), jnp.float32)

    def zb_body(i, _):
        zb_v[i, pl.ds(0, 16)] = z16
        return 0
    lax.fori_loop(0, 640, zb_body, 0)
    rbase = s * 3200
    for k in range(5):
        pltpu.sync_copy(zb_v, aggr.at[pl.ds(rbase + k * 640, 640)])
    plsc.subcore_barrier()

    base = wid * 195 + jnp.minimum(wid, 10)
    cnt = 195 + (wid < 10).astype(jnp.int32)

    def chunk(i, _):
        row = base + i
        pltpu.sync_copy(eidx.at[0].at[row], src_v)
        pltpu.sync_copy(eidx.at[1].at[row], dst_v)
        pltpu.sync_copy(eattr.at[row], ea_v)
        pltpu.async_copy(table.at[src_v], rows_v, sem).wait()

        def edge(g, _):
            ea0g = ea_v[0, pl.ds(g * 16, 16)]
            ea1g = ea_v[1, pl.ds(g * 16, 16)]
            for l in range(16):
                j = g * 16 + l
                r0 = rows_v[j, pl.ds(0, 16)]
                m0 = jnp.maximum(r0 + ea0g[l] * w0 + ea1g[l] * w1 + bb, 0.0)
                rows_v[j, pl.ds(0, 16)] = m0
            return 0
        lax.fori_loop(0, 8, edge, 0)
        pltpu.sync_copy(rows_v, aggr.at[dst_v], add=True)
        return 0
    lax.fori_loop(0, cnt, chunk, 0)
    plsc.subcore_barrier()

    for k in range(5):
        pltpu.sync_copy(aggr.at[pl.ds(rbase + k * 640, 640)],
                        out.at[pl.ds(c * NP + rbase + k * 640, 640)])


def _sc_msg32_body(table, eidx, eattr, wpk, out,
                   src_v, dst_v, ea_v, rows_v, w_v, zb_v, aggr, sem):
    c = lax.axis_index("c")
    s = lax.axis_index("s")

    pltpu.sync_copy(wpk.at[c], w_v)
    w0a = w_v[0, pl.ds(0, 16)]
    w0b = w_v[0, pl.ds(16, 16)]
    w1a = w_v[1, pl.ds(0, 16)]
    w1b = w_v[1, pl.ds(16, 16)]
    ba = w_v[2, pl.ds(0, 16)]
    bbb = w_v[2, pl.ds(16, 16)]
    z16 = jnp.zeros((16,), jnp.float32)

    def zb_body(i, _):
        zb_v[i, pl.ds(0, 16)] = z16
        zb_v[i, pl.ds(16, 16)] = z16
        return 0
    lax.fori_loop(0, 640, zb_body, 0)
    rbase = s * 3200
    for k in range(5):
        pltpu.sync_copy(zb_v, aggr.at[pl.ds(rbase + k * 640, 640)])
    plsc.subcore_barrier()

    base = s * 390 + jnp.minimum(s, 10)
    cnt = 390 + (s < 10).astype(jnp.int32)
    cN = c * NP

    def chunk(i, _):
        row = base + i
        pltpu.sync_copy(eidx.at[0].at[row], src_v)
        pltpu.sync_copy(eidx.at[1].at[row], dst_v)
        pltpu.sync_copy(eattr.at[row], ea_v)
        for k in range(8):
            src_v[pl.ds(k * 16, 16)] = src_v[pl.ds(k * 16, 16)] + cN
        pltpu.async_copy(table.at[src_v], rows_v, sem).wait()

        def edge(g, _):
            ea0g = ea_v[0, pl.ds(g * 16, 16)]
            ea1g = ea_v[1, pl.ds(g * 16, 16)]
            for l in range(16):
                j = g * 16 + l
                r0 = rows_v[j, pl.ds(0, 16)]
                r1 = rows_v[j, pl.ds(16, 16)]
                rows_v[j, pl.ds(0, 16)] = jnp.maximum(
                    r0 + ea0g[l] * w0a + ea1g[l] * w1a + ba, 0.0)
                rows_v[j, pl.ds(16, 16)] = jnp.maximum(
                    r1 + ea0g[l] * w0b + ea1g[l] * w1b + bbb, 0.0)
            return 0
        lax.fori_loop(0, 8, edge, 0)
        pltpu.sync_copy(rows_v, aggr.at[dst_v], add=True)
        return 0
    lax.fori_loop(0, cnt, chunk, 0)
    plsc.subcore_barrier()

    for k in range(5):
        pltpu.sync_copy(aggr.at[pl.ds(rbase + k * 640, 640)],
                        out.at[pl.ds(c * NP + rbase + k * 640, 640)])


def _sc_msg16(x16, eidx, eattr, wpk):
    return pl.kernel(
        _sc_msg16_body,
        out_type=jax.ShapeDtypeStruct((2 * NP, 16), jnp.float32),
        mesh=_mesh,
        compiler_params=pltpu.CompilerParams(use_tc_tiling_on_sc=False),
        scratch_types=[
            pltpu.VMEM((128,), jnp.int32),
            pltpu.VMEM((128,), jnp.int32),
            pltpu.VMEM((2, 128), jnp.float32),
            pltpu.VMEM((128, 16), jnp.float32),
            pltpu.VMEM((3, 16), jnp.float32),
            pltpu.VMEM((640, 16), jnp.float32),
            pltpu.VMEM_SHARED((NP, 16), jnp.float32),
            pltpu.SemaphoreType.DMA,
        ],
    )(x16, eidx, eattr, wpk)


def _sc_msg32(table, eidx, eattr, wpk):
    return pl.kernel(
        _sc_msg32_body,
        out_type=jax.ShapeDtypeStruct((2 * NP, 32), jnp.float32),
        mesh=_mesh,
        compiler_params=pltpu.CompilerParams(use_tc_tiling_on_sc=False),
        scratch_types=[
            pltpu.VMEM((128,), jnp.int32),
            pltpu.VMEM((128,), jnp.int32),
            pltpu.VMEM((2, 128), jnp.float32),
            pltpu.VMEM((128, 32), jnp.float32),
            pltpu.VMEM((3, 32), jnp.float32),
            pltpu.VMEM((640, 32), jnp.float32),
            pltpu.VMEM_SHARED((NP, 32), jnp.float32),
            pltpu.SemaphoreType.DMA,
        ],
    )(table, eidx, eattr, wpk)


def _denseA_l1_body(x_ref, a1_ref, a2_ref, W_ref, b_ref, t_ref, st_ref):
    i = pl.program_id(0)
    ag = a1_ref[0] + a2_ref[0]
    t = jnp.dot(x_ref[...] + ag, W_ref[...],
                preferred_element_type=jnp.float32) + b_ref[...]
    t_ref[...] = t
    rowid = lax.broadcasted_iota(jnp.int32, (BLK, 1), 0) + i * BLK
    m = (rowid < N).astype(jnp.float32)
    tm = t * m

    @pl.when(i == 0)
    def _():
        st_ref[...] = jnp.zeros_like(st_ref)
    st_ref[0:1, :] += jnp.sum(tm, axis=0, keepdims=True)
    st_ref[1:2, :] += jnp.sum(tm * tm, axis=0, keepdims=True)


def _denseA_body(x1_ref, x2_ref, a1_ref, a2_ref, W_ref, b_ref, t_ref, st_ref):
    i = pl.program_id(0)
    xin = jnp.concatenate([x1_ref[0], x2_ref[0]], axis=1)
    ag = jnp.concatenate([a1_ref[0], a2_ref[0]], axis=1)
    t = jnp.dot(xin + ag, W_ref[...],
                preferred_element_type=jnp.float32) + b_ref[...]
    t_ref[...] = t
    rowid = lax.broadcasted_iota(jnp.int32, (BLK, 1), 0) + i * BLK
    m = (rowid < N).astype(jnp.float32)
    tm = t * m

    @pl.when(i == 0)
    def _():
        st_ref[...] = jnp.zeros_like(st_ref)
    st_ref[0:1, :] += jnp.sum(tm, axis=0, keepdims=True)
    st_ref[1:2, :] += jnp.sum(tm * tm, axis=0, keepdims=True)


def _denseA_l1(x16, p, W, b):
    return pl.pallas_call(
        _denseA_l1_body,
        grid=(NBLK,),
        in_specs=[
            pl.BlockSpec((BLK, 16), lambda i: (i, 0)),
            pl.BlockSpec((1, BLK, 16), lambda i: (0, i, 0)),
            pl.BlockSpec((1, BLK, 16), lambda i: (1, i, 0)),
            pl.BlockSpec((16, 64), lambda i: (0, 0)),
            pl.BlockSpec((1, 64), lambda i: (0, 0)),
        ],
        out_specs=[
            pl.BlockSpec((BLK, 64), lambda i: (i, 0)),
            pl.BlockSpec((8, 64), lambda i: (0, 0)),
        ],
        out_shape=[
            jax.ShapeDtypeStruct((NP, 64), jnp.float32),
            jax.ShapeDtypeStruct((8, 64), jnp.float32),
        ],
    )(x16, p, p, W, b)


def _denseA(h, a, W, b):
    return pl.pallas_call(
        _denseA_body,
        grid=(NBLK,),
        in_specs=[
            pl.BlockSpec((1, BLK, 32), lambda i: (0, i, 0)),
            pl.BlockSpec((1, BLK, 32), lambda i: (1, i, 0)),
            pl.BlockSpec((1, BLK, 32), lambda i: (0, i, 0)),
            pl.BlockSpec((1, BLK, 32), lambda i: (1, i, 0)),
            pl.BlockSpec((64, 64), lambda i: (0, 0)),
            pl.BlockSpec((1, 64), lambda i: (0, 0)),
        ],
        out_specs=[
            pl.BlockSpec((BLK, 64), lambda i: (i, 0)),
            pl.BlockSpec((8, 64), lambda i: (0, 0)),
        ],
        out_shape=[
            jax.ShapeDtypeStruct((NP, 64), jnp.float32),
            jax.ShapeDtypeStruct((8, 64), jnp.float32),
        ],
    )(h, h, a, a, W, b)


def _bn_apply(t, st, g, be):
    mu = st[0:1, :] * (1.0 / N)
    ex2 = st[1:2, :] * (1.0 / N)
    var = ex2 - mu * mu
    inv = lax.rsqrt(var + BN_EPS)
    return jnp.maximum((t - mu) * inv * g + be, 0.0)


def _denseB_l1_body(t_ref, st_ref, g_ref, be_ref, x_ref, Wr_ref, br_ref, out_ref):
    hb = _bn_apply(t_ref[...], st_ref[...], g_ref[...], be_ref[...])
    res = jnp.dot(x_ref[...], Wr_ref[...],
                  preferred_element_type=jnp.float32) + br_ref[...]
    h = hb + res
    out_ref[0] = h[:, 0:32]
    out_ref[1] = h[:, 32:64]


def _denseB_body(t_ref, st_ref, g_ref, be_ref, r1_ref, r2_ref, out_ref):
    hb = _bn_apply(t_ref[...], st_ref[...], g_ref[...], be_ref[...])
    res = jnp.concatenate([r1_ref[0], r2_ref[0]], axis=1)
    h = hb + res
    out_ref[0] = h[:, 0:32]
    out_ref[1] = h[:, 32:64]


def _denseB_l3_body(t_ref, st_ref, g_ref, be_ref, r1_ref, r2_ref, out_ref):
    hb = _bn_apply(t_ref[...], st_ref[...], g_ref[...], be_ref[...])
    res = jnp.concatenate([r1_ref[0], r2_ref[0]], axis=1)
    out_ref[...] = hb + res


def _denseB_l1(t, st, g, be, x16, Wr, br):
    return pl.pallas_call(
        _denseB_l1_body,
        grid=(NBLK,),
        in_specs=[
            pl.BlockSpec((BLK, 64), lambda i: (i, 0)),
            pl.BlockSpec((8, 64), lambda i: (0, 0)),
            pl.BlockSpec((1, 64), lambda i: (0, 0)),
            pl.BlockSpec((1, 64), lambda i: (0, 0)),
            pl.BlockSpec((BLK, 16), lambda i: (i, 0)),
            pl.BlockSpec((16, 64), lambda i: (0, 0)),
            pl.BlockSpec((1, 64), lambda i: (0, 0)),
        ],
        out_specs=pl.BlockSpec((2, BLK, 32), lambda i: (0, i, 0)),
        out_shape=jax.ShapeDtypeStruct((2, NP, 32), jnp.float32),
    )(t, st, g, be, x16, Wr, br)


def _denseB(t, st, g, be, r):
    return pl.pallas_call(
        _denseB_body,
        grid=(NBLK,),
        in_specs=[
            pl.BlockSpec((BLK, 64), lambda i: (i, 0)),
            pl.BlockSpec((8, 64), lambda i: (0, 0)),
            pl.BlockSpec((1, 64), lambda i: (0, 0)),
            pl.BlockSpec((1, 64), lambda i: (0, 0)),
            pl.BlockSpec((1, BLK, 32), lambda i: (0, i, 0)),
            pl.BlockSpec((1, BLK, 32), lambda i: (1, i, 0)),
        ],
        out_specs=pl.BlockSpec((2, BLK, 32), lambda i: (0, i, 0)),
        out_shape=jax.ShapeDtypeStruct((2, NP, 32), jnp.float32),
    )(t, st, g, be, r, r)


def _denseB_l3(t, st, g, be, r):
    return pl.pallas_call(
        _denseB_l3_body,
        grid=(NBLK,),
        in_specs=[
            pl.BlockSpec((BLK, 64), lambda i: (i, 0)),
            pl.BlockSpec((8, 64), lambda i: (0, 0)),
            pl.BlockSpec((1, 64), lambda i: (0, 0)),
            pl.BlockSpec((1, 64), lambda i: (0, 0)),
            pl.BlockSpec((1, BLK, 32), lambda i: (0, i, 0)),
            pl.BlockSpec((1, BLK, 32), lambda i: (1, i, 0)),
        ],
        out_specs=pl.BlockSpec((BLK, 64), lambda i: (i, 0)),
        out_shape=jax.ShapeDtypeStruct((NP, 64), jnp.float32),
    )(t, st, g, be, r, r)


def _sc_pool(h, batch_pad):
    def body(h_hbm, b_hbm, ps_o, pm_o, pc_o, hv, bv, ps_v, pm_v, pc_v, sem):
        c = lax.axis_index("c")
        s = lax.axis_index("s")
        wid = s * 2 + c
        z16 = jnp.zeros((16,), jnp.float32)
        ninf = jnp.full((16,), -jnp.inf, jnp.float32)

        e0 = jnp.where(lax.broadcasted_iota(jnp.int32, (16,), 0) == 0, 1.0, 0.0)

        def init(i, _):
            for k in range(4):
                ps_v[i, pl.ds(k * 16, 16)] = z16
                pm_v[i, pl.ds(k * 16, 16)] = ninf
            pc_v[i, pl.ds(0, 16)] = z16
            return 0
        lax.fori_loop(0, GP, init, 0)

        base = wid * 12 + jnp.minimum(wid, 16)
        cnt = 12 + (wid < 16).astype(jnp.int32)

        def chunk(i, _):
            row0 = (base + i) * 128
            pltpu.sync_copy(h_hbm.at[pl.ds(row0, 128)], hv)
            pltpu.sync_copy(b_hbm.at[pl.ds(row0, 128)], bv)

            def rowf(g, _):
                gv = bv[pl.ds(g * 16, 16)]
                for l in range(16):
                    j = g * 16 + l
                    gid = gv[l]
                    for k in range(4):
                        hk = hv[j, pl.ds(k * 16, 16)]
                        ps_v[gid, pl.ds(k * 16, 16)] = ps_v[gid, pl.ds(k * 16, 16)] + hk
                        pm_v[gid, pl.ds(k * 16, 16)] = jnp.maximum(
                            pm_v[gid, pl.ds(k * 16, 16)], hk)
                    pc_v[gid, pl.ds(0, 16)] = pc_v[gid, pl.ds(0, 16)] + e0
                return 0
            lax.fori_loop(0, 8, rowf, 0)
            return 0
        lax.fori_loop(0, cnt, chunk, 0)

        pltpu.sync_copy(ps_v, ps_o.at[wid])
        pltpu.sync_copy(pm_v, pm_o.at[wid])
        pltpu.sync_copy(pc_v, pc_o.at[wid])

    return pl.kernel(
        body,
        out_type=[
            jax.ShapeDtypeStruct((32, GP, 64), jnp.float32),
            jax.ShapeDtypeStruct((32, GP, 64), jnp.float32),
            jax.ShapeDtypeStruct((32, GP, 16), jnp.float32),
        ],
        mesh=_mesh,
        compiler_params=pltpu.CompilerParams(use_tc_tiling_on_sc=False),
        scratch_types=[
            pltpu.VMEM((128, 64), jnp.float32),
            pltpu.VMEM((128,), jnp.int32),
            pltpu.VMEM((GP, 64), jnp.float32),
            pltpu.VMEM((GP, 64), jnp.float32),
            pltpu.VMEM((GP, 16), jnp.float32),
            pltpu.SemaphoreType.DMA,
        ],
    )(h, batch_pad)


def _cls_body(ps_ref, pm_ref, pc_ref, w1_ref, b1_ref, w2_ref, b2_ref, out_ref):
    s = jnp.sum(ps_ref[...][:, 0:G, :], axis=0)
    m = jnp.max(pm_ref[...][:, 0:G, :], axis=0)
    cc = jnp.sum(pc_ref[...][:, 0:G, 0], axis=0)
    mean = s / jnp.maximum(cc, 1.0)[:, None]
    m = jnp.where(jnp.isfinite(m), m, 0.0)
    z = jnp.concatenate([mean, m], axis=1)
    h1 = jnp.maximum(jnp.dot(z, w1_ref[...], preferred_element_type=jnp.float32)
                     + b1_ref[...], 0.0)
    out_ref[...] = jnp.dot(h1, w2_ref[...],
                           preferred_element_type=jnp.float32) + b2_ref[...]


def _classifier(ps, pm, pc, Wk1, bk1, Wk2, bk2):
    return pl.pallas_call(
        _cls_body,
        out_shape=jax.ShapeDtypeStruct((G, 10), jnp.float32),
    )(ps, pm, pc, Wk1, bk1.reshape(1, -1), Wk2, bk2.reshape(1, -1))


def kernel(x, edge_index, edge_attr, batch,
           We1, be1, We2, be2, We3, be3,
           Wc1, bc1, g1, t1, Wc2, bc2, g2, t2, Wc3, bc3, g3, t3,
           Wr, br, Wk1, bk1, Wk2, bk2):
    f32 = jnp.float32
    x16 = jnp.zeros((NP, 16), f32).at[:N, :12].set(x)
    eidx = edge_index.reshape(2, ER, 128)
    ea3 = edge_attr.T.reshape(2, ER, 128).transpose(1, 0, 2)
    batch_pad = jnp.concatenate([batch, jnp.full((NP - N,), G, jnp.int32)])

    w1pk = jnp.zeros((3, 16), f32).at[0:2, 0:12].set(We1).at[2, 0:12].set(be1)
    w2pk = jnp.stack([
        jnp.stack([We2[0, 0:32], We2[1, 0:32], be2[0:32]]),
        jnp.stack([We2[0, 32:64], We2[1, 32:64], be2[32:64]]),
    ])
    w3pk = jnp.stack([
        jnp.stack([We3[0, 0:32], We3[1, 0:32], be3[0:32]]),
        jnp.stack([We3[0, 32:64], We3[1, 32:64], be3[32:64]]),
    ])
    Wc1p = jnp.zeros((16, 64), f32).at[0:12, :].set(Wc1)
    Wrp = jnp.zeros((16, 64), f32).at[0:12, :].set(Wr)

    p1 = _sc_msg16(x16, eidx, ea3, w1pk).reshape(2, NP, 16)
    t1a, st1 = _denseA_l1(x16, p1, Wc1p, bc1.reshape(1, -1))
    h1 = _denseB_l1(t1a, st1, g1.reshape(1, -1), t1.reshape(1, -1),
                    x16, Wrp, br.reshape(1, -1))

    a2 = _sc_msg32(h1.reshape(2 * NP, 32), eidx, ea3, w2pk).reshape(2, NP, 32)
    t2a, st2 = _denseA(h1, a2, Wc2, bc2.reshape(1, -1))
    h2 = _denseB(t2a, st2, g2.reshape(1, -1), t2.reshape(1, -1), h1)

    a3 = _sc_msg32(h2.reshape(2 * NP, 32), eidx, ea3, w3pk).reshape(2, NP, 32)
    t3a, st3 = _denseA(h2, a3, Wc3, bc3.reshape(1, -1))
    h3 = _denseB_l3(t3a, st3, g3.reshape(1, -1), t3.reshape(1, -1), h2)

    ps, pm, pc = _sc_pool(h3, batch_pad)
    return _classifier(ps, pm, pc, Wk1, bk1, Wk2, bk2)


# speedup vs baseline: 6.6772x; 2.0660x over previous
import functools

import jax
import jax.numpy as jnp
from jax import lax
from jax.experimental import pallas as pl
from jax.experimental.pallas import tpu as pltpu
from jax.experimental.pallas import tpu_sc as plsc

N = 50000
NP = 51200
E = 800000
ER = E // 128
G = 256
GP = 272
BN_EPS = 1e-5
BLK = 1024
NBLK = NP // BLK

_mesh = plsc.VectorSubcoreMesh(core_axis_name="c", subcore_axis_name="s")


def _sc_msg16_body(table, eidx, eattr, wpk, out,
                   src_v, dst_v, ea_v, rows_v, w_v, zb_v, aggr,
                   sem_i, sem_g, sem_s):
    c = lax.axis_index("c")
    s = lax.axis_index("s")
    wid = s * 2 + c

    pltpu.sync_copy(wpk, w_v)
    w0 = w_v[0, pl.ds(0, 16)]
    w1 = w_v[1, pl.ds(0, 16)]
    bb = w_v[2, pl.ds(0, 16)]
    z16 = jnp.zeros((16,), jnp.float32)

    def zb_body(i, _):
        zb_v[i, pl.ds(0, 16)] = z16
        return 0
    lax.fori_loop(0, 128, zb_body, 0)
    rbase = s * 3200
    for k in range(25):
        pltpu.sync_copy(zb_v, aggr.at[pl.ds(rbase + k * 128, 128)])
    plsc.subcore_barrier()

    base = wid * 196

    def idx_cps(i, b):
        row = base + i * 2
        return [
            pltpu.make_async_copy(eidx.at[0].at[pl.ds(row, 2)],
                                  src_v.at[pl.ds(b * 2, 2)], sem_i),
            pltpu.make_async_copy(eidx.at[1].at[pl.ds(row, 2)],
                                  dst_v.at[pl.ds(b * 2, 2)], sem_i),
            pltpu.make_async_copy(eattr.at[pl.ds(row, 2)],
                                  ea_v.at[pl.ds(b * 2, 2)], sem_i),
        ]

    def gather_cps(b):
        return [pltpu.make_async_copy(table.at[src_v.at[b * 2 + r]],
                                      rows_v.at[b].at[pl.ds(r * 128, 128)],
                                      sem_g) for r in range(2)]

    def scatter_cps(b):
        return [pltpu.make_async_copy(rows_v.at[b].at[pl.ds(r * 128, 128)],
                                      aggr.at[dst_v.at[b * 2 + r]],
                                      sem_s) for r in range(2)]

    def compute(b):
        def edge(g, _):
            er = b * 2 + (g >> 3)
            eo = (g & 7) * 16
            ea0g = ea_v[er, 0, pl.ds(eo, 16)]
            ea1g = ea_v[er, 1, pl.ds(eo, 16)]
            for l in range(16):
                j = g * 16 + l
                r0 = rows_v[b, j, pl.ds(0, 16)]
                rows_v[b, j, pl.ds(0, 16)] = jnp.maximum(
                    r0 + ea0g[l] * w0 + ea1g[l] * w1 + bb, 0.0)
            return 0
        lax.fori_loop(0, 16, edge, 0)

    for cp in idx_cps(0, 0):
        cp.start()

    def pair(p, _):
        i0 = 2 * p
        for cp in idx_cps(i0, 0):
            cp.wait()
        for cp in gather_cps(0):
            cp.start()

        @pl.when(p > 0)
        def _():
            for cp in scatter_cps(1):
                cp.wait()
        for cp in idx_cps(i0 + 1, 1):
            cp.start()
        for cp in gather_cps(0):
            cp.wait()
        compute(0)
        for cp in scatter_cps(0):
            cp.start(add=True)
        for cp in idx_cps(i0 + 1, 1):
            cp.wait()
        for cp in gather_cps(1):
            cp.start()
        for cp in scatter_cps(0):
            cp.wait()

        @pl.when(p < 48)
        def _():
            for cp in idx_cps(i0 + 2, 0):
                cp.start()
        for cp in gather_cps(1):
            cp.wait()
        compute(1)
        for cp in scatter_cps(1):
            cp.start(add=True)
        return 0
    lax.fori_loop(0, 49, pair, 0)
    for cp in scatter_cps(1):
        cp.wait()
    plsc.subcore_barrier()

    for k in range(5):
        pltpu.sync_copy(aggr.at[pl.ds(rbase + k * 640, 640)],
                        out.at[pl.ds(c * NP + rbase + k * 640, 640)])


def _sc_msg32_body(table, eidx, eattr, wpk, out,
                   src_v, dst_v, ea_v, rows_v, w_v, zb_v, aggr,
                   sem_i, sem_g, sem_s):
    c = lax.axis_index("c")
    s = lax.axis_index("s")

    pltpu.sync_copy(wpk.at[c], w_v)
    w0a = w_v[0, pl.ds(0, 16)]
    w0b = w_v[0, pl.ds(16, 16)]
    w1a = w_v[1, pl.ds(0, 16)]
    w1b = w_v[1, pl.ds(16, 16)]
    ba = w_v[2, pl.ds(0, 16)]
    bbb = w_v[2, pl.ds(16, 16)]
    z16 = jnp.zeros((16,), jnp.float32)

    def zb_body(i, _):
        zb_v[i, pl.ds(0, 16)] = z16
        zb_v[i, pl.ds(16, 16)] = z16
        return 0
    lax.fori_loop(0, 128, zb_body, 0)
    rbase = s * 3200
    for k in range(25):
        pltpu.sync_copy(zb_v, aggr.at[pl.ds(rbase + k * 128, 128)])
    plsc.subcore_barrier()

    base = s * 392
    cN = c * NP

    def idx_cps(i, b):
        row = base + i * 2
        return [
            pltpu.make_async_copy(eidx.at[0].at[pl.ds(row, 2)],
                                  src_v.at[pl.ds(b * 2, 2)], sem_i),
            pltpu.make_async_copy(eidx.at[1].at[pl.ds(row, 2)],
                                  dst_v.at[pl.ds(b * 2, 2)], sem_i),
            pltpu.make_async_copy(eattr.at[pl.ds(row, 2)],
                                  ea_v.at[pl.ds(b * 2, 2)], sem_i),
        ]

    def gather_cps(b):
        return [pltpu.make_async_copy(table.at[src_v.at[b * 2 + r]],
                                      rows_v.at[b].at[pl.ds(r * 128, 128)],
                                      sem_g) for r in range(2)]

    def scatter_cps(b):
        return [pltpu.make_async_copy(rows_v.at[b].at[pl.ds(r * 128, 128)],
                                      aggr.at[dst_v.at[b * 2 + r]],
                                      sem_s) for r in range(2)]

    def bump_src(b):
        for r in range(2):
            for k in range(8):
                src_v[b * 2 + r, pl.ds(k * 16, 16)] = (
                    src_v[b * 2 + r, pl.ds(k * 16, 16)] + cN)

    def compute(b):
        def edge(g, _):
            er = b * 2 + (g >> 3)
            eo = (g & 7) * 16
            ea0g = ea_v[er, 0, pl.ds(eo, 16)]
            ea1g = ea_v[er, 1, pl.ds(eo, 16)]
            for l in range(16):
                j = g * 16 + l
                r0 = rows_v[b, j, pl.ds(0, 16)]
                r1 = rows_v[b, j, pl.ds(16, 16)]
                rows_v[b, j, pl.ds(0, 16)] = jnp.maximum(
                    r0 + ea0g[l] * w0a + ea1g[l] * w1a + ba, 0.0)
                rows_v[b, j, pl.ds(16, 16)] = jnp.maximum(
                    r1 + ea0g[l] * w0b + ea1g[l] * w1b + bbb, 0.0)
            return 0
        lax.fori_loop(0, 16, edge, 0)

    for cp in idx_cps(0, 0):
        cp.start()

    def pair(p, _):
        i0 = 2 * p
        for cp in idx_cps(i0, 0):
            cp.wait()
        bump_src(0)
        for cp in gather_cps(0):
            cp.start()

        @pl.when(p > 0)
        def _():
            for cp in scatter_cps(1):
                cp.wait()
        for cp in idx_cps(i0 + 1, 1):
            cp.start()
        for cp in gather_cps(0):
            cp.wait()
        compute(0)
        for cp in scatter_cps(0):
            cp.start(add=True)
        for cp in idx_cps(i0 + 1, 1):
            cp.wait()
        bump_src(1)
        for cp in gather_cps(1):
            cp.start()
        for cp in scatter_cps(0):
            cp.wait()

        @pl.when(p < 97)
        def _():
            for cp in idx_cps(i0 + 2, 0):
                cp.start()
        for cp in gather_cps(1):
            cp.wait()
        compute(1)
        for cp in scatter_cps(1):
            cp.start(add=True)
        return 0
    lax.fori_loop(0, 98, pair, 0)
    for cp in scatter_cps(1):
        cp.wait()
    plsc.subcore_barrier()

    for k in range(5):
        pltpu.sync_copy(aggr.at[pl.ds(rbase + k * 640, 640)],
                        out.at[pl.ds(c * NP + rbase + k * 640, 640)])


def _sc_msg16(x16, eidx, eattr, wpk):
    return pl.kernel(
        _sc_msg16_body,
        out_type=jax.ShapeDtypeStruct((2 * NP, 16), jnp.float32),
        mesh=_mesh,
        compiler_params=pltpu.CompilerParams(use_tc_tiling_on_sc=False),
        scratch_types=[
            pltpu.VMEM((4, 128), jnp.int32),
            pltpu.VMEM((4, 128), jnp.int32),
            pltpu.VMEM((4, 2, 128), jnp.float32),
            pltpu.VMEM((2, 256, 16), jnp.float32),
            pltpu.VMEM((3, 16), jnp.float32),
            pltpu.VMEM((128, 16), jnp.float32),
            pltpu.VMEM_SHARED((NP, 16), jnp.float32),
            pltpu.SemaphoreType.DMA,
            pltpu.SemaphoreType.DMA,
            pltpu.SemaphoreType.DMA,
        ],
    )(x16, eidx, eattr, wpk)


def _sc_msg32(table, eidx, eattr, wpk):
    return pl.kernel(
        _sc_msg32_body,
        out_type=jax.ShapeDtypeStruct((2 * NP, 32), jnp.float32),
        mesh=_mesh,
        compiler_params=pltpu.CompilerParams(use_tc_tiling_on_sc=False),
        scratch_types=[
            pltpu.VMEM((4, 128), jnp.int32),
            pltpu.VMEM((4, 128), jnp.int32),
            pltpu.VMEM((4, 2, 128), jnp.float32),
            pltpu.VMEM((2, 256, 32), jnp.float32),
            pltpu.VMEM((3, 32), jnp.float32),
            pltpu.VMEM((128, 32), jnp.float32),
            pltpu.VMEM_SHARED((NP, 32), jnp.float32),
            pltpu.SemaphoreType.DMA,
            pltpu.SemaphoreType.DMA,
            pltpu.SemaphoreType.DMA,
        ],
    )(table, eidx, eattr, wpk)


def _denseA_l1_body(x_ref, a1_ref, a2_ref, W_ref, b_ref, t_ref, st_ref):
    i = pl.program_id(0)
    ag = a1_ref[0] + a2_ref[0]
    t = jnp.dot(x_ref[...] + ag, W_ref[...],
                preferred_element_type=jnp.float32) + b_ref[...]
    t_ref[...] = t
    rowid = lax.broadcasted_iota(jnp.int32, (BLK, 1), 0) + i * BLK
    m = (rowid < N).astype(jnp.float32)
    tm = t * m

    @pl.when(i == 0)
    def _():
        st_ref[...] = jnp.zeros_like(st_ref)
    st_ref[0:1, :] += jnp.sum(tm, axis=0, keepdims=True)
    st_ref[1:2, :] += jnp.sum(tm * tm, axis=0, keepdims=True)


def _denseA_body(x1_ref, x2_ref, a1_ref, a2_ref, W_ref, b_ref, t_ref, st_ref):
    i = pl.program_id(0)
    xin = jnp.concatenate([x1_ref[0], x2_ref[0]], axis=1)
    ag = jnp.concatenate([a1_ref[0], a2_ref[0]], axis=1)
    t = jnp.dot(xin + ag, W_ref[...],
                preferred_element_type=jnp.float32) + b_ref[...]
    t_ref[...] = t
    rowid = lax.broadcasted_iota(jnp.int32, (BLK, 1), 0) + i * BLK
    m = (rowid < N).astype(jnp.float32)
    tm = t * m

    @pl.when(i == 0)
    def _():
        st_ref[...] = jnp.zeros_like(st_ref)
    st_ref[0:1, :] += jnp.sum(tm, axis=0, keepdims=True)
    st_ref[1:2, :] += jnp.sum(tm * tm, axis=0, keepdims=True)


def _denseA_l1(x16, p, W, b):
    return pl.pallas_call(
        _denseA_l1_body,
        grid=(NBLK,),
        in_specs=[
            pl.BlockSpec((BLK, 16), lambda i: (i, 0)),
            pl.BlockSpec((1, BLK, 16), lambda i: (0, i, 0)),
            pl.BlockSpec((1, BLK, 16), lambda i: (1, i, 0)),
            pl.BlockSpec((16, 64), lambda i: (0, 0)),
            pl.BlockSpec((1, 64), lambda i: (0, 0)),
        ],
        out_specs=[
            pl.BlockSpec((BLK, 64), lambda i: (i, 0)),
            pl.BlockSpec((8, 64), lambda i: (0, 0)),
        ],
        out_shape=[
            jax.ShapeDtypeStruct((NP, 64), jnp.float32),
            jax.ShapeDtypeStruct((8, 64), jnp.float32),
        ],
    )(x16, p, p, W, b)


def _denseA(h, a, W, b):
    return pl.pallas_call(
        _denseA_body,
        grid=(NBLK,),
        in_specs=[
            pl.BlockSpec((1, BLK, 32), lambda i: (0, i, 0)),
            pl.BlockSpec((1, BLK, 32), lambda i: (1, i, 0)),
            pl.BlockSpec((1, BLK, 32), lambda i: (0, i, 0)),
            pl.BlockSpec((1, BLK, 32), lambda i: (1, i, 0)),
            pl.BlockSpec((64, 64), lambda i: (0, 0)),
            pl.BlockSpec((1, 64), lambda i: (0, 0)),
        ],
        out_specs=[
            pl.BlockSpec((BLK, 64), lambda i: (i, 0)),
            pl.BlockSpec((8, 64), lambda i: (0, 0)),
        ],
        out_shape=[
            jax.ShapeDtypeStruct((NP, 64), jnp.float32),
            jax.ShapeDtypeStruct((8, 64), jnp.float32),
        ],
    )(h, h, a, a, W, b)


def _bn_apply(t, st, g, be):
    mu = st[0:1, :] * (1.0 / N)
    ex2 = st[1:2, :] * (1.0 / N)
    var = ex2 - mu * mu
    inv = lax.rsqrt(var + BN_EPS)
    return jnp.maximum((t - mu) * inv * g + be, 0.0)


def _denseB_l1_body(t_ref, st_ref, g_ref, be_ref, x_ref, Wr_ref, br_ref, out_ref):
    hb = _bn_apply(t_ref[...], st_ref[...], g_ref[...], be_ref[...])
    res = jnp.dot(x_ref[...], Wr_ref[...],
                  preferred_element_type=jnp.float32) + br_ref[...]
    h = hb + res
    out_ref[0] = h[:, 0:32]
    out_ref[1] = h[:, 32:64]


def _denseB_body(t_ref, st_ref, g_ref, be_ref, r1_ref, r2_ref, out_ref):
    hb = _bn_apply(t_ref[...], st_ref[...], g_ref[...], be_ref[...])
    res = jnp.concatenate([r1_ref[0], r2_ref[0]], axis=1)
    h = hb + res
    out_ref[0] = h[:, 0:32]
    out_ref[1] = h[:, 32:64]


def _denseB_l3_body(t_ref, st_ref, g_ref, be_ref, r1_ref, r2_ref, out_ref):
    hb = _bn_apply(t_ref[...], st_ref[...], g_ref[...], be_ref[...])
    res = jnp.concatenate([r1_ref[0], r2_ref[0]], axis=1)
    out_ref[...] = hb + res


def _denseB_l1(t, st, g, be, x16, Wr, br):
    return pl.pallas_call(
        _denseB_l1_body,
        grid=(NBLK,),
        in_specs=[
            pl.BlockSpec((BLK, 64), lambda i: (i, 0)),
            pl.BlockSpec((8, 64), lambda i: (0, 0)),
            pl.BlockSpec((1, 64), lambda i: (0, 0)),
            pl.BlockSpec((1, 64), lambda i: (0, 0)),
            pl.BlockSpec((BLK, 16), lambda i: (i, 0)),
            pl.BlockSpec((16, 64), lambda i: (0, 0)),
            pl.BlockSpec((1, 64), lambda i: (0, 0)),
        ],
        out_specs=pl.BlockSpec((2, BLK, 32), lambda i: (0, i, 0)),
        out_shape=jax.ShapeDtypeStruct((2, NP, 32), jnp.float32),
    )(t, st, g, be, x16, Wr, br)


def _denseB(t, st, g, be, r):
    return pl.pallas_call(
        _denseB_body,
        grid=(NBLK,),
        in_specs=[
            pl.BlockSpec((BLK, 64), lambda i: (i, 0)),
            pl.BlockSpec((8, 64), lambda i: (0, 0)),
            pl.BlockSpec((1, 64), lambda i: (0, 0)),
            pl.BlockSpec((1, 64), lambda i: (0, 0)),
            pl.BlockSpec((1, BLK, 32), lambda i: (0, i, 0)),
            pl.BlockSpec((1, BLK, 32), lambda i: (1, i, 0)),
        ],
        out_specs=pl.BlockSpec((2, BLK, 32), lambda i: (0, i, 0)),
        out_shape=jax.ShapeDtypeStruct((2, NP, 32), jnp.float32),
    )(t, st, g, be, r, r)


def _denseB_l3(t, st, g, be, r):
    return pl.pallas_call(
        _denseB_l3_body,
        grid=(NBLK,),
        in_specs=[
            pl.BlockSpec((BLK, 64), lambda i: (i, 0)),
            pl.BlockSpec((8, 64), lambda i: (0, 0)),
            pl.BlockSpec((1, 64), lambda i: (0, 0)),
            pl.BlockSpec((1, 64), lambda i: (0, 0)),
            pl.BlockSpec((1, BLK, 32), lambda i: (0, i, 0)),
            pl.BlockSpec((1, BLK, 32), lambda i: (1, i, 0)),
        ],
        out_specs=pl.BlockSpec((BLK, 64), lambda i: (i, 0)),
        out_shape=jax.ShapeDtypeStruct((NP, 64), jnp.float32),
    )(t, st, g, be, r, r)


def _sc_pool(h, batch_pad):
    def body(h_hbm, b_hbm, ps_o, pm_o, pc_o, hv, bv, ps_v, pm_v, pc_v, sem):
        c = lax.axis_index("c")
        s = lax.axis_index("s")
        wid = s * 2 + c
        z16 = jnp.zeros((16,), jnp.float32)
        ninf = jnp.full((16,), -jnp.inf, jnp.float32)

        e0 = jnp.where(lax.broadcasted_iota(jnp.int32, (16,), 0) == 0, 1.0, 0.0)

        def init(i, _):
            for k in range(4):
                ps_v[i, pl.ds(k * 16, 16)] = z16
                pm_v[i, pl.ds(k * 16, 16)] = ninf
            pc_v[i, pl.ds(0, 16)] = z16
            return 0
        lax.fori_loop(0, GP, init, 0)

        base = wid * 12 + jnp.minimum(wid, 16)
        cnt = 12 + (wid < 16).astype(jnp.int32)

        def chunk(i, _):
            row0 = (base + i) * 128
            pltpu.sync_copy(h_hbm.at[pl.ds(row0, 128)], hv)
            pltpu.sync_copy(b_hbm.at[pl.ds(row0, 128)], bv)

            def rowf(g, _):
                gv = bv[pl.ds(g * 16, 16)]
                for l in range(16):
                    j = g * 16 + l
                    gid = gv[l]
                    for k in range(4):
                        hk = hv[j, pl.ds(k * 16, 16)]
                        ps_v[gid, pl.ds(k * 16, 16)] = ps_v[gid, pl.ds(k * 16, 16)] + hk
                        pm_v[gid, pl.ds(k * 16, 16)] = jnp.maximum(
                            pm_v[gid, pl.ds(k * 16, 16)], hk)
                    pc_v[gid, pl.ds(0, 16)] = pc_v[gid, pl.ds(0, 16)] + e0
                return 0
            lax.fori_loop(0, 8, rowf, 0)
            return 0
        lax.fori_loop(0, cnt, chunk, 0)

        pltpu.sync_copy(ps_v, ps_o.at[wid])
        pltpu.sync_copy(pm_v, pm_o.at[wid])
        pltpu.sync_copy(pc_v, pc_o.at[wid])

    return pl.kernel(
        body,
        out_type=[
            jax.ShapeDtypeStruct((32, GP, 64), jnp.float32),
            jax.ShapeDtypeStruct((32, GP, 64), jnp.float32),
            jax.ShapeDtypeStruct((32, GP, 16), jnp.float32),
        ],
        mesh=_mesh,
        compiler_params=pltpu.CompilerParams(use_tc_tiling_on_sc=False),
        scratch_types=[
            pltpu.VMEM((128, 64), jnp.float32),
            pltpu.VMEM((128,), jnp.int32),
            pltpu.VMEM((GP, 64), jnp.float32),
            pltpu.VMEM((GP, 64), jnp.float32),
            pltpu.VMEM((GP, 16), jnp.float32),
            pltpu.SemaphoreType.DMA,
        ],
    )(h, batch_pad)


def _cls_body(ps_ref, pm_ref, pc_ref, w1_ref, b1_ref, w2_ref, b2_ref, out_ref):
    s = jnp.sum(ps_ref[...][:, 0:G, :], axis=0)
    m = jnp.max(pm_ref[...][:, 0:G, :], axis=0)
    cc = jnp.sum(pc_ref[...][:, 0:G, 0], axis=0)
    mean = s / jnp.maximum(cc, 1.0)[:, None]
    m = jnp.where(jnp.isfinite(m), m, 0.0)
    z = jnp.concatenate([mean, m], axis=1)
    h1 = jnp.maximum(jnp.dot(z, w1_ref[...], preferred_element_type=jnp.float32)
                     + b1_ref[...], 0.0)
    out_ref[...] = jnp.dot(h1, w2_ref[...],
                           preferred_element_type=jnp.float32) + b2_ref[...]


def _classifier(ps, pm, pc, Wk1, bk1, Wk2, bk2):
    return pl.pallas_call(
        _cls_body,
        out_shape=jax.ShapeDtypeStruct((G, 10), jnp.float32),
    )(ps, pm, pc, Wk1, bk1.reshape(1, -1), Wk2, bk2.reshape(1, -1))


def kernel(x, edge_index, edge_attr, batch,
           We1, be1, We2, be2, We3, be3,
           Wc1, bc1, g1, t1, Wc2, bc2, g2, t2, Wc3, bc3, g3, t3,
           Wr, br, Wk1, bk1, Wk2, bk2):
    f32 = jnp.float32
    x16 = jnp.zeros((NP, 16), f32).at[:N, :12].set(x)
    eidx = jnp.concatenate([
        edge_index.reshape(2, ER, 128),
        jnp.stack([jnp.zeros((22, 128), jnp.int32),
                   jnp.full((22, 128), N, jnp.int32)]),
    ], axis=1)
    ea3 = jnp.concatenate([
        edge_attr.T.reshape(2, ER, 128).transpose(1, 0, 2),
        jnp.zeros((22, 2, 128), f32),
    ], axis=0)
    batch_pad = jnp.concatenate([batch, jnp.full((NP - N,), G, jnp.int32)])

    w1pk = jnp.zeros((3, 16), f32).at[0:2, 0:12].set(We1).at[2, 0:12].set(be1)
    w2pk = jnp.stack([
        jnp.stack([We2[0, 0:32], We2[1, 0:32], be2[0:32]]),
        jnp.stack([We2[0, 32:64], We2[1, 32:64], be2[32:64]]),
    ])
    w3pk = jnp.stack([
        jnp.stack([We3[0, 0:32], We3[1, 0:32], be3[0:32]]),
        jnp.stack([We3[0, 32:64], We3[1, 32:64], be3[32:64]]),
    ])
    Wc1p = jnp.zeros((16, 64), f32).at[0:12, :].set(Wc1)
    Wrp = jnp.zeros((16, 64), f32).at[0:12, :].set(Wr)

    p1 = _sc_msg16(x16, eidx, ea3, w1pk).reshape(2, NP, 16)
    t1a, st1 = _denseA_l1(x16, p1, Wc1p, bc1.reshape(1, -1))
    h1 = _denseB_l1(t1a, st1, g1.reshape(1, -1), t1.reshape(1, -1),
                    x16, Wrp, br.reshape(1, -1))

    a2 = _sc_msg32(h1.reshape(2 * NP, 32), eidx, ea3, w2pk).reshape(2, NP, 32)
    t2a, st2 = _denseA(h1, a2, Wc2, bc2.reshape(1, -1))
    h2 = _denseB(t2a, st2, g2.reshape(1, -1), t2.reshape(1, -1), h1)

    a3 = _sc_msg32(h2.reshape(2 * NP, 32), eidx, ea3, w3pk).reshape(2, NP, 32)
    t3a, st3 = _denseA(h2, a3, Wc3, bc3.reshape(1, -1))
    h3 = _denseB_l3(t3a, st3, g3.reshape(1, -1), t3.reshape(1, -1), h2)

    ps, pm, pc = _sc_pool(h3, batch_pad)
    return _classifier(ps, pm, pc, Wk1, bk1, Wk2, bk2)
